# baseline (speedup 1.0000x reference)
import jax
import jax.numpy as jnp
from jax import lax
from jax.experimental import pallas as pl
from jax.experimental.pallas import tpu as pltpu
from jax.experimental.pallas import tpu_sc as plsc

N = 10000
NP = 10240
E = 320000
D = 128
NT = 32
FR = D // NT
CB = 2048
EP = 327680
NB = EP // CB
HB = NB // 2
MASK = 65535


def _make_sc_t():
  mesh = plsc.VectorSubcoreMesh(core_axis_name="c", subcore_axis_name="s")
  out_type = [jax.ShapeDtypeStruct((NT, FR, NP), jnp.float32)]
  scratch = (
      [pltpu.VMEM((NP,), jnp.float32)] * FR +
      [pltpu.VMEM((NP,), jnp.float32)] * FR +
      [pltpu.VMEM((2, CB), jnp.int32),
       pltpu.SemaphoreType.DMA,
       pltpu.SemaphoreType.DMA]
  )

  def body(zt_hbm, pk_hbm, z0_hbm, out_hbm, *refs):
    zts = refs[:FR]
    ats = refs[FR:2 * FR]
    pkt, sem0, sem1 = refs[2 * FR:]
    c = lax.axis_index("c")
    s = lax.axis_index("s")
    tid = c * (NT // 2) + s
    for f in range(FR):
      pltpu.sync_copy(zt_hbm.at[tid, f], zts[f])
      pltpu.sync_copy(z0_hbm.at[f], ats[f])

    def process(slot):
      def vec(v, carry):
        pv = pkt[slot, pl.ds(v * 16, 16)]
        sidx = jnp.bitwise_and(pv, MASK)
        didx = lax.shift_right_logical(pv, 16)
        for f in range(FR):
          val = plsc.load_gather(zts[f], [sidx])
          plsc.addupdate_scatter(ats[f], [didx], val)
        return carry

      lax.fori_loop(0, CB // 16, vec, 0)

    pltpu.async_copy(pk_hbm.at[pl.ds(0, CB)], pkt.at[0], sem0)

    def step(t, carry):
      g0 = 2 * t
      g1 = g0 + 1
      pltpu.async_copy(pk_hbm.at[pl.ds(g1 * CB, CB)], pkt.at[1], sem1)
      pltpu.make_async_copy(pk_hbm.at[pl.ds(g0 * CB, CB)], pkt.at[0],
                            sem0).wait()
      process(0)

      @pl.when(t < HB - 1)
      def _():
        pltpu.async_copy(pk_hbm.at[pl.ds((g0 + 2) * CB, CB)], pkt.at[0],
                         sem0)

      pltpu.make_async_copy(pk_hbm.at[pl.ds(g1 * CB, CB)], pkt.at[1],
                            sem1).wait()
      process(1)
      return carry

    lax.fori_loop(0, HB, step, 0)
    for f in range(FR):
      pltpu.sync_copy(ats[f], out_hbm.at[tid, f])

  return pl.kernel(
      body, mesh=mesh, out_type=out_type, scratch_types=scratch,
      compiler_params=pltpu.CompilerParams(needs_layout_passes=False))


def _mm(w_ref, h):
  return lax.dot_general(w_ref[...], h, (((1,), (0,)), ((), ())),
                         preferred_element_type=jnp.float32)


_FULL = lambda *shape: pl.BlockSpec(shape, lambda i: (0,) * len(shape))


def _tc_pre(xp, wl, wr):
  def body(x_ref, wl_ref, wr_ref, z_ref, r_ref):
    z_ref[...] = lax.dot_general(wl_ref[...], x_ref[...],
                                 (((1,), (1,)), ((), ())),
                                 preferred_element_type=jnp.float32)
    r_ref[...] = lax.dot_general(wr_ref[...], x_ref[...],
                                 (((1,), (1,)), ((), ())),
                                 preferred_element_type=jnp.float32)

  return pl.pallas_call(
      body,
      grid=(1,),
      in_specs=[_FULL(NP, D), _FULL(D, D), _FULL(D, D)],
      out_specs=[_FULL(D, NP), _FULL(D, NP)],
      out_shape=[jax.ShapeDtypeStruct((D, NP), jnp.float32)] * 2,
  )(xp, wl, wr)


def _make_tc_update(final):
  def body(*refs):
    agg, dgt, r, b = refs[:4]
    rd = 1.0 / jnp.maximum(dgt[0:1, :], 1.0)
    h = jnp.maximum(agg[...] * rd + r[...] + b[...], 0.0)
    if final:
      o_ref = refs[4]
      o_ref[...] = h
    else:
      wl, wr, z_ref, rn_ref = refs[4:]
      z_ref[...] = _mm(wl, h)
      rn_ref[...] = _mm(wr, h)

  in_specs = [_FULL(D, NP), pl.BlockSpec((8, NP), lambda i: (0, 0)),
              _FULL(D, NP), _FULL(D, 1)]
  if final:
    out_specs = [_FULL(D, NP)]
    out_shape = [jax.ShapeDtypeStruct((D, NP), jnp.float32)]
  else:
    in_specs += [_FULL(D, D), _FULL(D, D)]
    out_specs = [_FULL(D, NP)] * 2
    out_shape = [jax.ShapeDtypeStruct((D, NP), jnp.float32)] * 2

  return pl.pallas_call(body, grid=(1,), in_specs=in_specs,
                        out_specs=out_specs, out_shape=out_shape)


def kernel(x, edge_index, Wl1, Wr1, b1, Wl2, Wr2, b2, Wl3, Wr3, b3,
           Wl4, Wr4, b4, Wl5, Wr5, b5):
  pad = EP - E
  src_p = jnp.concatenate([edge_index[0], jnp.zeros((pad,), jnp.int32)])
  dst_p = jnp.concatenate([edge_index[1], jnp.full((pad,), N, jnp.int32)])
  pk = jnp.bitwise_or(src_p, lax.shift_left(dst_p, jnp.int32(16)))

  xp = jnp.zeros((NP, D), jnp.float32).at[:N].set(x)
  onest = jnp.ones((NT, FR, NP), jnp.float32)
  zeros4 = jnp.zeros((FR, NP), jnp.float32)
  bcol = lambda b: b.reshape(D, 1)
  wl5p = jnp.zeros((D, D), jnp.float32).at[:17].set(Wl5)
  wr5p = jnp.zeros((D, D), jnp.float32).at[:17].set(Wr5)
  b5c = jnp.zeros((D, 1), jnp.float32).at[:17, 0].set(b5)

  sc = _make_sc_t()
  up = _make_tc_update(False)
  (degt,) = sc(onest, pk, zeros4)
  degt = degt.reshape(D, NP)

  zt, rt = _tc_pre(xp, Wl1, Wr1)
  (agg,) = sc(zt.reshape(NT, FR, NP), pk, zeros4)
  zt, rt = up(agg.reshape(D, NP), degt, rt, bcol(b1), Wl2, Wr2)
  (agg,) = sc(zt.reshape(NT, FR, NP), pk, zeros4)
  zt, rt = up(agg.reshape(D, NP), degt, rt, bcol(b2), Wl3, Wr3)
  (agg,) = sc(zt.reshape(NT, FR, NP), pk, zeros4)
  zt, rt = up(agg.reshape(D, NP), degt, rt, bcol(b3), Wl4, Wr4)
  (agg,) = sc(zt.reshape(NT, FR, NP), pk, zeros4)
  zt, rt = up(agg.reshape(D, NP), degt, rt, bcol(b4), wl5p, wr5p)
  (agg,) = sc(zt.reshape(NT, FR, NP), pk, zeros4)
  (outt,) = _make_tc_update(True)(agg.reshape(D, NP), degt, rt, b5c)
  return outt[:17, :N].T
